# Initial kernel scaffold; baseline (speedup 1.0000x reference)
import jax
import jax.numpy as jnp
from jax.experimental import pallas as pl


def kernel(s_factor, image, projection_data, tof_value, x1l, y1l, x1r, y1r, x2l, y2l, x2r, y2r):
    raise NotImplementedError("write your pallas kernel here")


# speedup vs baseline: 294.3127x; 294.3127x over previous
import dataclasses
import functools

import jax
import jax.numpy as jnp
from jax import lax
from jax.experimental import pallas as pl
from jax.experimental.pallas import tpu as pltpu
from jax.experimental.pallas import tpu_sc as plsc

NX = 256
NY = 256
NXY = NX * NY
SIGMA = 30.0
N_SAMPLES = 32
EVENT_NUM = 400000

NW = 32
CHUNK = 512
NCHUNK = 25
EPW = CHUNK * NCHUNK
EPAD = NW * EPW
NROW = 8
GAUSS_A = -0.5 / (SIGMA * SIGMA)


def _corner_data(fx, fy):
    xt = fx.astype(jnp.int32)
    xtf = xt.astype(jnp.float32)
    x0 = jnp.where(fx < xtf, xt - 1, xt)
    yt = fy.astype(jnp.int32)
    ytf = yt.astype(jnp.float32)
    y0 = jnp.where(fy < ytf, yt - 1, yt)
    x1 = x0 + 1
    y1 = y0 + 1
    wx1 = fx - x0.astype(jnp.float32)
    wx0 = 1.0 - wx1
    wy1 = fy - y0.astype(jnp.float32)
    wy0 = 1.0 - wy1
    zero = jnp.zeros_like(fx)
    vx0 = (x0 >= 0) & (x0 < NX)
    vx1 = (x1 >= 0) & (x1 < NX)
    vy0 = (y0 >= 0) & (y0 < NY)
    vy1 = (y1 >= 0) & (y1 < NY)
    x0c = jnp.minimum(jnp.maximum(x0, 0), NX - 1) * NY
    x1c = jnp.minimum(jnp.maximum(x1, 0), NX - 1) * NY
    y0c = jnp.minimum(jnp.maximum(y0, 0), NY - 1)
    y1c = jnp.minimum(jnp.maximum(y1, 0), NY - 1)
    return [
        (x0c + y0c, wx0 * wy0, vx0 & vy0, zero),
        (x0c + y1c, wx0 * wy1, vx0 & vy1, zero),
        (x1c + y0c, wx1 * wy0, vx1 & vy0, zero),
        (x1c + y1c, wx1 * wy1, vx1 & vy1, zero),
    ]


def _sc_kernel(img_hbm, ev_hbm, out_hbm, img_v, ev_v, diff_v, sem):
    wid = lax.axis_index("s") * 2 + lax.axis_index("c")

    pltpu.async_copy(img_hbm, img_v, sem).wait()

    @pl.loop(0, NCHUNK)
    def _chunk_p(c):
        pltpu.async_copy(ev_hbm.at[wid, c], ev_v, sem).wait()

        @pl.loop(0, CHUNK // 16)
        def _vec_p(v):
            sl = pl.ds(v * 16, 16)
            p1x = ev_v[0, sl]
            p1y = ev_v[1, sl]
            dxl = ev_v[2, sl]
            dyl = ev_v[3, sl]
            ll = ev_v[4, sl]
            uu = ev_v[5, sl]
            step = ev_v[6, sl]
            pdat = ev_v[7, sl]
            acc = jnp.zeros((16,), jnp.float32)
            for j in range(N_SAMPLES):
                t = jnp.float32(j / (N_SAMPLES - 1.0))
                fx = p1x + t * dxl
                fy = p1y + t * dyl
                d = t * ll - uu
                w = jnp.exp((GAUSS_A * d) * d)
                csum = jnp.zeros((16,), jnp.float32)
                for idx, cw, valid, zero in _corner_data(fx, fy):
                    g = plsc.load_gather(img_v, [idx])
                    csum = csum + jnp.where(valid, cw, zero) * g
                acc = acc + csum * w
            diff_v[pl.ds(c * CHUNK + v * 16, 16)] = (acc * step - pdat) * step

    @pl.loop(0, NXY, step=16)
    def _zero(i):
        img_v[pl.ds(i, 16)] = jnp.zeros((16,), jnp.float32)

    @pl.loop(0, NCHUNK)
    def _chunk_b(c):
        pltpu.async_copy(ev_hbm.at[wid, c], ev_v, sem).wait()

        @pl.loop(0, CHUNK // 16)
        def _vec_b(v):
            sl = pl.ds(v * 16, 16)
            p1x = ev_v[0, sl]
            p1y = ev_v[1, sl]
            dxl = ev_v[2, sl]
            dyl = ev_v[3, sl]
            ll = ev_v[4, sl]
            uu = ev_v[5, sl]
            dv = diff_v[pl.ds(c * CHUNK + v * 16, 16)]
            for j in range(N_SAMPLES):
                t = jnp.float32(j / (N_SAMPLES - 1.0))
                fx = p1x + t * dxl
                fy = p1y + t * dyl
                d = t * ll - uu
                w = jnp.exp((GAUSS_A * d) * d)
                val = dv * w
                for idx, cw, valid, zero in _corner_data(fx, fy):
                    plsc.addupdate_scatter(img_v, [idx], cw * val, mask=valid)

    pltpu.async_copy(img_v, out_hbm.at[wid], sem).wait()


@jax.jit
def _sc_call(img_flat, ev):
    mesh = plsc.VectorSubcoreMesh(core_axis_name="c", subcore_axis_name="s")
    cp = pltpu.CompilerParams()
    if "needs_layout_passes" in pltpu.CompilerParams.__dataclass_fields__:
        cp = dataclasses.replace(cp, needs_layout_passes=False)
    f = functools.partial(
        pl.kernel,
        out_type=jax.ShapeDtypeStruct((NW, NXY), jnp.float32),
        mesh=mesh,
        scratch_types=[
            pltpu.VMEM((NXY,), jnp.float32),
            pltpu.VMEM((NROW, CHUNK), jnp.float32),
            pltpu.VMEM((EPW,), jnp.float32),
            pltpu.SemaphoreType.DMA,
        ],
        compiler_params=cp,
    )(_sc_kernel)
    return f(img_flat, ev)


def _combine_body(s_ref, acc_ref, img_ref, o_ref):
    o_ref[...] = img_ref[...] - s_ref[0] * jnp.sum(acc_ref[...], axis=0)


@jax.jit
def _combine(s_factor, accs, img_flat):
    blk = 4096
    return pl.pallas_call(
        _combine_body,
        grid=(NXY // blk,),
        in_specs=[
            pl.BlockSpec(memory_space=pltpu.SMEM),
            pl.BlockSpec((NW, blk), lambda i: (0, i)),
            pl.BlockSpec((blk,), lambda i: (i,)),
        ],
        out_specs=pl.BlockSpec((blk,), lambda i: (i,)),
        out_shape=jax.ShapeDtypeStruct((NXY,), jnp.float32),
    )(s_factor, accs, img_flat)


def kernel(s_factor, image, projection_data, tof_value,
           x1l, y1l, x1r, y1r, x2l, y2l, x2r, y2r):
    p1x = 0.5 * (x1l + x1r)
    p1y = 0.5 * (y1l + y1r)
    p2x = 0.5 * (x2l + x2r)
    p2y = 0.5 * (y2l + y2r)
    dxl = p2x - p1x
    dyl = p2y - p1y
    ll = jnp.sqrt(dxl * dxl + dyl * dyl)
    step = ll / (N_SAMPLES - 1.0)
    uu = 0.5 * ll + tof_value
    rows = jnp.stack(
        [p1x + (NX / 2 - 0.5), p1y + (NY / 2 - 0.5), dxl, dyl, ll, uu, step,
         projection_data], 0)
    rows = jnp.pad(rows, ((0, 0), (0, EPAD - EVENT_NUM)))
    ev = rows.reshape(NROW, NW, NCHUNK, CHUNK).transpose(1, 2, 0, 3)
    img_flat = image.reshape(NXY)
    accs = _sc_call(img_flat, ev)
    out = _combine(s_factor, accs, img_flat)
    return out.reshape(1, 1, NX, NY)


# speedup vs baseline: 400.1354x; 1.3596x over previous
import dataclasses
import functools

import jax
import jax.numpy as jnp
from jax import lax
from jax.experimental import pallas as pl
from jax.experimental.pallas import tpu as pltpu
from jax.experimental.pallas import tpu_sc as plsc

NX = 256
NY = 256
NXP = NX + 2
NYP = NY + 2
IMGW = 66688
SIGMA = 30.0
N_SAMPLES = 32
EVENT_NUM = 400000
GAUSS_A = -0.5 / (SIGMA * SIGMA)

NW = 32
CHUNK = 512
NCHUNK = 25
EPW = CHUNK * NCHUNK
EPAD = NW * EPW
NROW = 9


def _corner_idx(fx, fy):
    fxc = jnp.minimum(jnp.maximum(fx, 0.0), float(NX) + 0.5)
    fyc = jnp.minimum(jnp.maximum(fy, 0.0), float(NY) + 0.5)
    x0 = fxc.astype(jnp.int32)
    y0 = fyc.astype(jnp.int32)
    wx1 = fxc - x0.astype(jnp.float32)
    wy1 = fyc - y0.astype(jnp.float32)
    wx0 = 1.0 - wx1
    wy0 = 1.0 - wy1
    i00 = x0 * NYP + y0
    i01 = i00 + 1
    i10 = i00 + NYP
    i11 = i00 + (NYP + 1)
    return ((i00, wx0 * wy0), (i01, wx0 * wy1),
            (i10, wx1 * wy0), (i11, wx1 * wy1))


def _sc_kernel(img_hbm, ev_hbm, out_hbm, img_v, ev_v, diff_v, sem):
    wid = lax.axis_index("s") * 2 + lax.axis_index("c")
    lanes_true = lax.iota(jnp.int32, 16) >= 0

    pltpu.async_copy(img_hbm, img_v, sem).wait()

    @pl.loop(0, NCHUNK)
    def _chunk_p(c):
        pltpu.async_copy(ev_hbm.at[wid, c], ev_v, sem).wait()

        @pl.loop(0, CHUNK // 16)
        def _vec_p(v):
            sl = pl.ds(v * 16, 16)
            fx0 = ev_v[0, sl]
            fy0 = ev_v[1, sl]
            dfx = ev_v[2, sl]
            dfy = ev_v[3, sl]
            w0 = ev_v[4, sl]
            r0 = ev_v[5, sl]
            qq = ev_v[6, sl]
            s2 = ev_v[7, sl]
            ps = ev_v[8, sl]

            def body(j, carry):
                fx, fy, w, r, acc = carry
                csum = None
                for idx, cw in _corner_idx(fx, fy):
                    g = cw * plsc.load_gather(img_v, [idx])
                    csum = g if csum is None else csum + g
                acc = acc + csum * w
                return (fx + dfx, fy + dfy, w * r, r * qq, acc)

            acc0 = jnp.zeros((16,), jnp.float32)
            acc = lax.fori_loop(0, N_SAMPLES, body,
                                (fx0, fy0, w0, r0, acc0))[4]
            diff_v[pl.ds(c * CHUNK + v * 16, 16)] = acc * s2 - ps

    @pl.loop(0, IMGW, step=16)
    def _zero(i):
        img_v[pl.ds(i, 16)] = jnp.zeros((16,), jnp.float32)

    @pl.loop(0, NCHUNK)
    def _chunk_b(c):
        pltpu.async_copy(ev_hbm.at[wid, c], ev_v, sem).wait()

        @pl.loop(0, CHUNK // 16)
        def _vec_b(v):
            sl = pl.ds(v * 16, 16)
            fx0 = ev_v[0, sl]
            fy0 = ev_v[1, sl]
            dfx = ev_v[2, sl]
            dfy = ev_v[3, sl]
            w0 = ev_v[4, sl]
            r0 = ev_v[5, sl]
            qq = ev_v[6, sl]
            dv = diff_v[pl.ds(c * CHUNK + v * 16, 16)]

            def body(j, carry):
                fx, fy, w, r = carry
                val = dv * w
                for idx, cw in _corner_idx(fx, fy):
                    plsc.addupdate_scatter(img_v, [idx], cw * val,
                                           mask=lanes_true)
                return (fx + dfx, fy + dfy, w * r, r * qq)

            lax.fori_loop(0, N_SAMPLES, body, (fx0, fy0, w0, r0))

    pltpu.async_copy(img_v, out_hbm.at[wid], sem).wait()


@jax.jit
def _sc_call(img_flat, ev):
    mesh = plsc.VectorSubcoreMesh(core_axis_name="c", subcore_axis_name="s")
    cp = pltpu.CompilerParams()
    if "needs_layout_passes" in pltpu.CompilerParams.__dataclass_fields__:
        cp = dataclasses.replace(cp, needs_layout_passes=False)
    f = functools.partial(
        pl.kernel,
        out_type=jax.ShapeDtypeStruct((NW, IMGW), jnp.float32),
        mesh=mesh,
        scratch_types=[
            pltpu.VMEM((IMGW,), jnp.float32),
            pltpu.VMEM((NROW, CHUNK), jnp.float32),
            pltpu.VMEM((EPW,), jnp.float32),
            pltpu.SemaphoreType.DMA,
        ],
        compiler_params=cp,
    )(_sc_kernel)
    return f(img_flat, ev)


def _combine_body(s_ref, acc_ref, img_ref, o_ref):
    o_ref[...] = img_ref[...] - s_ref[0] * jnp.sum(acc_ref[...], axis=0)


@jax.jit
def _combine(s_factor, accs, img_flat):
    return pl.pallas_call(
        _combine_body,
        in_specs=[
            pl.BlockSpec(memory_space=pltpu.SMEM),
            pl.BlockSpec((NW, IMGW), lambda: (0, 0)),
            pl.BlockSpec((IMGW,), lambda: (0,)),
        ],
        out_specs=pl.BlockSpec((IMGW,), lambda: (0,)),
        out_shape=jax.ShapeDtypeStruct((IMGW,), jnp.float32),
    )(s_factor, accs, img_flat)


def kernel(s_factor, image, projection_data, tof_value,
           x1l, y1l, x1r, y1r, x2l, y2l, x2r, y2r):
    p1x = 0.5 * (x1l + x1r)
    p1y = 0.5 * (y1l + y1r)
    p2x = 0.5 * (x2l + x2r)
    p2y = 0.5 * (y2l + y2r)
    dxl = p2x - p1x
    dyl = p2y - p1y
    ll = jnp.sqrt(dxl * dxl + dyl * dyl)
    step = ll / (N_SAMPLES - 1.0)
    uu = 0.5 * ll + tof_value
    astep = GAUSS_A * step
    w0 = jnp.exp((GAUSS_A * uu) * uu)
    r0 = jnp.exp(astep * (step - 2.0 * uu))
    qq = jnp.exp(2.0 * astep * step)
    bias = NX / 2 - 0.5 + 1.0
    rows = jnp.stack(
        [p1x + bias, p1y + bias, dxl / (N_SAMPLES - 1.0),
         dyl / (N_SAMPLES - 1.0), w0, r0, qq, step * step,
         projection_data * step], 0)
    rows = jnp.pad(rows, ((0, 0), (0, EPAD - EVENT_NUM)))
    ev = rows.reshape(NROW, NW, NCHUNK, CHUNK).transpose(1, 2, 0, 3)
    img_flat = jnp.pad(image[0, 0], ((1, 1), (1, 1))).reshape(-1)
    img_flat = jnp.pad(img_flat, (0, IMGW - NXP * NYP))
    accs = _sc_call(img_flat, ev)
    out = _combine(s_factor, accs, img_flat)
    out = out[:NXP * NYP].reshape(NXP, NYP)[1:-1, 1:-1]
    return out.reshape(1, 1, NX, NY)


# speedup vs baseline: 405.2746x; 1.0128x over previous
import dataclasses
import functools

import jax
import jax.numpy as jnp
from jax import lax
from jax.experimental import pallas as pl
from jax.experimental.pallas import tpu as pltpu
from jax.experimental.pallas import tpu_sc as plsc

NX = 256
NY = 256
NXP = NX + 2
NYP = NY + 2
IMGW = 66688
SIGMA = 30.0
N_SAMPLES = 32
EVENT_NUM = 400000
GAUSS_A = -0.5 / (SIGMA * SIGMA)

NW = 32
CHUNK = 512
NCHUNK = 25
EPW = CHUNK * NCHUNK
EPAD = NW * EPW
NROW = 9


def _corner_idx(fx, fy):
    fxc = jnp.minimum(jnp.maximum(fx, 0.0), float(NX) + 0.5)
    fyc = jnp.minimum(jnp.maximum(fy, 0.0), float(NY) + 0.5)
    x0 = fxc.astype(jnp.int32)
    y0 = fyc.astype(jnp.int32)
    wx1 = fxc - x0.astype(jnp.float32)
    wy1 = fyc - y0.astype(jnp.float32)
    wx0 = 1.0 - wx1
    wy0 = 1.0 - wy1
    i00 = x0 * NYP + y0
    i01 = i00 + 1
    i10 = i00 + NYP
    i11 = i00 + (NYP + 1)
    return ((i00, wx0 * wy0), (i01, wx0 * wy1),
            (i10, wx1 * wy0), (i11, wx1 * wy1))


def _sc_kernel(img_hbm, ev_hbm, out_hbm, img_v, ev_v, diff_v, sem):
    wid = lax.axis_index("s") * 2 + lax.axis_index("c")
    lanes_true = lax.iota(jnp.int32, 16) >= 0

    pltpu.async_copy(img_hbm, img_v, sem).wait()

    @pl.loop(0, NCHUNK)
    def _chunk_p(c):
        pltpu.async_copy(ev_hbm.at[wid, c], ev_v, sem).wait()

        @plsc.parallel_loop(0, CHUNK // 16, unroll=2)
        def _vec_p(v):
            sl = pl.ds(v * 16, 16)
            fx0 = ev_v[0, sl]
            fy0 = ev_v[1, sl]
            dfx = ev_v[2, sl]
            dfy = ev_v[3, sl]
            w0 = ev_v[4, sl]
            r0 = ev_v[5, sl]
            qq = ev_v[6, sl]
            s2 = ev_v[7, sl]
            ps = ev_v[8, sl]

            def body(j, carry):
                fx, fy, w, r, acc = carry
                csum = None
                for idx, cw in _corner_idx(fx, fy):
                    g = cw * plsc.load_gather(img_v, [idx])
                    csum = g if csum is None else csum + g
                acc = acc + csum * w
                return (fx + dfx, fy + dfy, w * r, r * qq, acc)

            acc0 = jnp.zeros((16,), jnp.float32)
            acc = lax.fori_loop(0, N_SAMPLES, body,
                                (fx0, fy0, w0, r0, acc0))[4]
            diff_v[pl.ds(c * CHUNK + v * 16, 16)] = acc * s2 - ps

    @plsc.parallel_loop(0, IMGW, step=16, unroll=4)
    def _zero(i):
        img_v[pl.ds(i, 16)] = jnp.zeros((16,), jnp.float32)

    @pl.loop(0, NCHUNK)
    def _chunk_b(c):
        pltpu.async_copy(ev_hbm.at[wid, c], ev_v, sem).wait()

        @plsc.parallel_loop(0, CHUNK // 16, unroll=2)
        def _vec_b(v):
            sl = pl.ds(v * 16, 16)
            fx0 = ev_v[0, sl]
            fy0 = ev_v[1, sl]
            dfx = ev_v[2, sl]
            dfy = ev_v[3, sl]
            w0 = ev_v[4, sl]
            r0 = ev_v[5, sl]
            qq = ev_v[6, sl]
            dv = diff_v[pl.ds(c * CHUNK + v * 16, 16)]

            def body(j, carry):
                fx, fy, w, r = carry
                val = dv * w
                for idx, cw in _corner_idx(fx, fy):
                    plsc.addupdate_scatter(img_v, [idx], cw * val,
                                           mask=lanes_true)
                return (fx + dfx, fy + dfy, w * r, r * qq)

            lax.fori_loop(0, N_SAMPLES, body, (fx0, fy0, w0, r0))

    pltpu.async_copy(img_v, out_hbm.at[wid], sem).wait()


@jax.jit
def _sc_call(img_flat, ev):
    mesh = plsc.VectorSubcoreMesh(core_axis_name="c", subcore_axis_name="s")
    cp = pltpu.CompilerParams()
    if "needs_layout_passes" in pltpu.CompilerParams.__dataclass_fields__:
        cp = dataclasses.replace(cp, needs_layout_passes=False)
    f = functools.partial(
        pl.kernel,
        out_type=jax.ShapeDtypeStruct((NW, IMGW), jnp.float32),
        mesh=mesh,
        scratch_types=[
            pltpu.VMEM((IMGW,), jnp.float32),
            pltpu.VMEM((NROW, CHUNK), jnp.float32),
            pltpu.VMEM((EPW,), jnp.float32),
            pltpu.SemaphoreType.DMA,
        ],
        compiler_params=cp,
    )(_sc_kernel)
    return f(img_flat, ev)


def _combine_body(s_ref, acc_ref, img_ref, o_ref):
    o_ref[...] = img_ref[...] - s_ref[0] * jnp.sum(acc_ref[...], axis=0)


@jax.jit
def _combine(s_factor, accs, img_flat):
    return pl.pallas_call(
        _combine_body,
        in_specs=[
            pl.BlockSpec(memory_space=pltpu.SMEM),
            pl.BlockSpec((NW, IMGW), lambda: (0, 0)),
            pl.BlockSpec((IMGW,), lambda: (0,)),
        ],
        out_specs=pl.BlockSpec((IMGW,), lambda: (0,)),
        out_shape=jax.ShapeDtypeStruct((IMGW,), jnp.float32),
    )(s_factor, accs, img_flat)


def kernel(s_factor, image, projection_data, tof_value,
           x1l, y1l, x1r, y1r, x2l, y2l, x2r, y2r):
    p1x = 0.5 * (x1l + x1r)
    p1y = 0.5 * (y1l + y1r)
    p2x = 0.5 * (x2l + x2r)
    p2y = 0.5 * (y2l + y2r)
    dxl = p2x - p1x
    dyl = p2y - p1y
    ll = jnp.sqrt(dxl * dxl + dyl * dyl)
    step = ll / (N_SAMPLES - 1.0)
    uu = 0.5 * ll + tof_value
    astep = GAUSS_A * step
    w0 = jnp.exp((GAUSS_A * uu) * uu)
    r0 = jnp.exp(astep * (step - 2.0 * uu))
    qq = jnp.exp(2.0 * astep * step)
    bias = NX / 2 - 0.5 + 1.0
    rows = jnp.stack(
        [p1x + bias, p1y + bias, dxl / (N_SAMPLES - 1.0),
         dyl / (N_SAMPLES - 1.0), w0, r0, qq, step * step,
         projection_data * step], 0)
    rows = jnp.pad(rows, ((0, 0), (0, EPAD - EVENT_NUM)))
    ev = rows.reshape(NROW, NW, NCHUNK, CHUNK).transpose(1, 2, 0, 3)
    img_flat = jnp.pad(image[0, 0], ((1, 1), (1, 1))).reshape(-1)
    img_flat = jnp.pad(img_flat, (0, IMGW - NXP * NYP))
    accs = _sc_call(img_flat, ev)
    out = _combine(s_factor, accs, img_flat)
    out = out[:NXP * NYP].reshape(NXP, NYP)[1:-1, 1:-1]
    return out.reshape(1, 1, NX, NY)


# speedup vs baseline: 406.1814x; 1.0022x over previous
import dataclasses
import functools

import jax
import jax.numpy as jnp
from jax import lax
from jax.experimental import pallas as pl
from jax.experimental.pallas import tpu as pltpu
from jax.experimental.pallas import tpu_sc as plsc

NX = 256
NY = 256
NXP = NX + 2
NYP = NY + 2
IMGW = 66688
SIGMA = 30.0
N_SAMPLES = 32
EVENT_NUM = 400000
GAUSS_A = -0.5 / (SIGMA * SIGMA)

NW = 32
CHUNK = 512
NCHUNK = 25
EPW = CHUNK * NCHUNK
EPAD = NW * EPW
NROW = 9


def _corner_idx(fx, fy):
    fxc = jnp.minimum(jnp.maximum(fx, 0.0), float(NX) + 0.5)
    fyc = jnp.minimum(jnp.maximum(fy, 0.0), float(NY) + 0.5)
    x0 = fxc.astype(jnp.int32)
    y0 = fyc.astype(jnp.int32)
    wx1 = fxc - x0.astype(jnp.float32)
    wy1 = fyc - y0.astype(jnp.float32)
    wx0 = 1.0 - wx1
    wy0 = 1.0 - wy1
    i00 = x0 * NYP + y0
    i01 = i00 + 1
    i10 = i00 + NYP
    i11 = i00 + (NYP + 1)
    return ((i00, wx0 * wy0), (i01, wx0 * wy1),
            (i10, wx1 * wy0), (i11, wx1 * wy1))


def _sc_kernel(img_hbm, ev_hbm, out_hbm, img_v, ev_v, diff_v, sem):
    wid = lax.axis_index("s") * 2 + lax.axis_index("c")
    lanes_true = lax.iota(jnp.int32, 16) >= 0

    pltpu.async_copy(img_hbm, img_v, sem).wait()

    @pl.loop(0, NCHUNK)
    def _chunk_p(c):
        pltpu.async_copy(ev_hbm.at[wid, c], ev_v, sem).wait()

        @plsc.parallel_loop(0, CHUNK // 16, unroll=2)
        def _vec_p(v):
            sl = pl.ds(v * 16, 16)
            fx0 = ev_v[0, sl]
            fy0 = ev_v[1, sl]
            dfx = ev_v[2, sl]
            dfy = ev_v[3, sl]
            w0 = ev_v[4, sl]
            r0 = ev_v[5, sl]
            qq = ev_v[6, sl]
            s2 = ev_v[7, sl]
            ps = ev_v[8, sl]

            def body(j, carry):
                fx, fy, w, r, acc = carry
                fxb = fx + dfx
                fyb = fy + dfy
                wb = w * r
                rb = r * qq
                csum = None
                for idx, cw in _corner_idx(fx, fy):
                    g = cw * plsc.load_gather(img_v, [idx])
                    csum = g if csum is None else csum + g
                csb = None
                for idx, cw in _corner_idx(fxb, fyb):
                    g = cw * plsc.load_gather(img_v, [idx])
                    csb = g if csb is None else csb + g
                acc = acc + csum * w + csb * wb
                return (fxb + dfx, fyb + dfy, wb * rb, rb * qq, acc)

            acc0 = jnp.zeros((16,), jnp.float32)
            acc = lax.fori_loop(0, N_SAMPLES // 2, body,
                                (fx0, fy0, w0, r0, acc0))[4]
            diff_v[pl.ds(c * CHUNK + v * 16, 16)] = acc * s2 - ps

    @plsc.parallel_loop(0, IMGW, step=16, unroll=4)
    def _zero(i):
        img_v[pl.ds(i, 16)] = jnp.zeros((16,), jnp.float32)

    @pl.loop(0, NCHUNK)
    def _chunk_b(c):
        pltpu.async_copy(ev_hbm.at[wid, c], ev_v, sem).wait()

        @plsc.parallel_loop(0, CHUNK // 16, unroll=2)
        def _vec_b(v):
            sl = pl.ds(v * 16, 16)
            fx0 = ev_v[0, sl]
            fy0 = ev_v[1, sl]
            dfx = ev_v[2, sl]
            dfy = ev_v[3, sl]
            w0 = ev_v[4, sl]
            r0 = ev_v[5, sl]
            qq = ev_v[6, sl]
            dv = diff_v[pl.ds(c * CHUNK + v * 16, 16)]

            def body(j, carry):
                fx, fy, w, r = carry
                fxb = fx + dfx
                fyb = fy + dfy
                wb = w * r
                rb = r * qq
                val = dv * w
                valb = dv * wb
                for idx, cw in _corner_idx(fx, fy):
                    plsc.addupdate_scatter(img_v, [idx], cw * val,
                                           mask=lanes_true)
                for idx, cw in _corner_idx(fxb, fyb):
                    plsc.addupdate_scatter(img_v, [idx], cw * valb,
                                           mask=lanes_true)
                return (fxb + dfx, fyb + dfy, wb * rb, rb * qq)

            lax.fori_loop(0, N_SAMPLES // 2, body, (fx0, fy0, w0, r0))

    pltpu.async_copy(img_v, out_hbm.at[wid], sem).wait()


@jax.jit
def _sc_call(img_flat, ev):
    mesh = plsc.VectorSubcoreMesh(core_axis_name="c", subcore_axis_name="s")
    cp = pltpu.CompilerParams()
    if "needs_layout_passes" in pltpu.CompilerParams.__dataclass_fields__:
        cp = dataclasses.replace(cp, needs_layout_passes=False)
    f = functools.partial(
        pl.kernel,
        out_type=jax.ShapeDtypeStruct((NW, IMGW), jnp.float32),
        mesh=mesh,
        scratch_types=[
            pltpu.VMEM((IMGW,), jnp.float32),
            pltpu.VMEM((NROW, CHUNK), jnp.float32),
            pltpu.VMEM((EPW,), jnp.float32),
            pltpu.SemaphoreType.DMA,
        ],
        compiler_params=cp,
    )(_sc_kernel)
    return f(img_flat, ev)


def _combine_body(s_ref, acc_ref, img_ref, o_ref):
    o_ref[...] = img_ref[...] - s_ref[0] * jnp.sum(acc_ref[...], axis=0)


@jax.jit
def _combine(s_factor, accs, img_flat):
    return pl.pallas_call(
        _combine_body,
        in_specs=[
            pl.BlockSpec(memory_space=pltpu.SMEM),
            pl.BlockSpec((NW, IMGW), lambda: (0, 0)),
            pl.BlockSpec((IMGW,), lambda: (0,)),
        ],
        out_specs=pl.BlockSpec((IMGW,), lambda: (0,)),
        out_shape=jax.ShapeDtypeStruct((IMGW,), jnp.float32),
    )(s_factor, accs, img_flat)


def kernel(s_factor, image, projection_data, tof_value,
           x1l, y1l, x1r, y1r, x2l, y2l, x2r, y2r):
    p1x = 0.5 * (x1l + x1r)
    p1y = 0.5 * (y1l + y1r)
    p2x = 0.5 * (x2l + x2r)
    p2y = 0.5 * (y2l + y2r)
    dxl = p2x - p1x
    dyl = p2y - p1y
    ll = jnp.sqrt(dxl * dxl + dyl * dyl)
    step = ll / (N_SAMPLES - 1.0)
    uu = 0.5 * ll + tof_value
    astep = GAUSS_A * step
    w0 = jnp.exp((GAUSS_A * uu) * uu)
    r0 = jnp.exp(astep * (step - 2.0 * uu))
    qq = jnp.exp(2.0 * astep * step)
    bias = NX / 2 - 0.5 + 1.0
    rows = jnp.stack(
        [p1x + bias, p1y + bias, dxl / (N_SAMPLES - 1.0),
         dyl / (N_SAMPLES - 1.0), w0, r0, qq, step * step,
         projection_data * step], 0)
    rows = jnp.pad(rows, ((0, 0), (0, EPAD - EVENT_NUM)))
    ev = rows.reshape(NROW, NW, NCHUNK, CHUNK).transpose(1, 2, 0, 3)
    img_flat = jnp.pad(image[0, 0], ((1, 1), (1, 1))).reshape(-1)
    img_flat = jnp.pad(img_flat, (0, IMGW - NXP * NYP))
    accs = _sc_call(img_flat, ev)
    out = _combine(s_factor, accs, img_flat)
    out = out[:NXP * NYP].reshape(NXP, NYP)[1:-1, 1:-1]
    return out.reshape(1, 1, NX, NY)
